# baseline (speedup 1.0000x reference)
import functools

import jax
import jax.numpy as jnp
from jax import lax
from jax.experimental import pallas as pl
from jax.experimental.pallas import tpu as pltpu
from jax.experimental.pallas import tpu_sc as plsc

_B = 4 * 4096
_D = 1024
_NC = 2
_NS = 16
_NW = _NC * _NS
_BPW = _B // _NW
_C = 32
_NCHUNK = _BPW // _C
_NBUF = 3
_GDEPTH = 2


def _emb_body(idx_hbm, table_hbm, out_hbm, idx_v, *rest):
    bufs = rest[:_NBUF]
    gsems = rest[_NBUF:2 * _NBUF]
    osems = rest[2 * _NBUF:3 * _NBUF]
    wid = lax.axis_index("s") * _NC + lax.axis_index("c")
    base = wid * _BPW
    pltpu.sync_copy(idx_hbm.at[wid], idx_v)

    ghandles = [None] * _NCHUNK
    ohandles = [None] * _NCHUNK

    def writeout(g):
        ghandles[g].wait()
        ohandles[g] = pltpu.async_copy(
            bufs[g % _NBUF], out_hbm.at[pl.ds(base + g * _C, _C)],
            osems[g % _NBUF])

    for c in range(_NCHUNK):
        if c >= _NBUF:
            ohandles[c - _NBUF].wait()
        ghandles[c] = pltpu.async_copy(
            table_hbm.at[idx_v.at[c]], bufs[c % _NBUF],
            gsems[c % _NBUF])
        if c >= _GDEPTH - 1:
            writeout(c - (_GDEPTH - 1))
    for g in range(_NCHUNK - (_GDEPTH - 1), _NCHUNK):
        writeout(g)
    for g in range(_NCHUNK - _NBUF, _NCHUNK):
        ohandles[g].wait()


@functools.partial(jax.jit, static_argnames=())
def kernel(input_ids, word_embeddings):
    idx = input_ids.reshape(_NW, _NCHUNK, _C).astype(jnp.int32)
    mesh = plsc.VectorSubcoreMesh(core_axis_name="c", subcore_axis_name="s")
    run = pl.kernel(
        _emb_body,
        out_type=jax.ShapeDtypeStruct((_B, _D), jnp.float32),
        mesh=mesh,
        scratch_types=(
            [pltpu.VMEM((_NCHUNK, _C), jnp.int32)]
            + [pltpu.VMEM((_C, _D), jnp.float32)] * _NBUF
            + [pltpu.SemaphoreType.DMA] * (2 * _NBUF)
        ),
    )
    out = run(idx, word_embeddings)
    return out.reshape(input_ids.shape + (_D,))


# speedup vs baseline: 1.0050x; 1.0050x over previous
import functools

import jax
import jax.numpy as jnp
from jax import lax
from jax.experimental import pallas as pl
from jax.experimental.pallas import tpu as pltpu
from jax.experimental.pallas import tpu_sc as plsc

_B = 4 * 4096
_D = 1024
_NC = 2
_NS = 16
_NW = _NC * _NS
_BPW = _B // _NW
_C = 32
_NCHUNK = _BPW // _C
_NBUF = 3
_GDEPTH = 2


def _emb_body(idx_hbm, table_hbm, out_hbm, idx_v, *rest):
    bufs = rest[:_NBUF]
    gsems = rest[_NBUF:2 * _NBUF]
    osems = rest[2 * _NBUF:3 * _NBUF]
    wid = lax.axis_index("s") * _NC + lax.axis_index("c")
    base = wid * _BPW
    pltpu.sync_copy(idx_hbm.at[wid // 8, pl.ds((wid % 8) * _BPW, _BPW)], idx_v)

    ghandles = [None] * _NCHUNK
    ohandles = [None] * _NCHUNK

    def writeout(g):
        ghandles[g].wait()
        ohandles[g] = pltpu.async_copy(
            bufs[g % _NBUF], out_hbm.at[pl.ds(base + g * _C, _C)],
            osems[g % _NBUF])

    for c in range(_NCHUNK):
        if c >= _NBUF:
            ohandles[c - _NBUF].wait()
        ghandles[c] = pltpu.async_copy(
            table_hbm.at[idx_v.at[pl.ds(c * _C, _C)]], bufs[c % _NBUF],
            gsems[c % _NBUF])
        if c >= _GDEPTH - 1:
            writeout(c - (_GDEPTH - 1))
    for g in range(_NCHUNK - (_GDEPTH - 1), _NCHUNK):
        writeout(g)
    for g in range(_NCHUNK - _NBUF, _NCHUNK):
        ohandles[g].wait()


@functools.partial(jax.jit, static_argnames=())
def kernel(input_ids, word_embeddings):
    idx = input_ids
    mesh = plsc.VectorSubcoreMesh(core_axis_name="c", subcore_axis_name="s")
    run = pl.kernel(
        _emb_body,
        out_type=jax.ShapeDtypeStruct((_B, _D), jnp.float32),
        mesh=mesh,
        scratch_types=(
            [pltpu.VMEM((_BPW,), jnp.int32)]
            + [pltpu.VMEM((_C, _D), jnp.float32)] * _NBUF
            + [pltpu.SemaphoreType.DMA] * (2 * _NBUF)
        ),
    )
    out = run(idx, word_embeddings)
    return out.reshape(input_ids.shape + (_D,))


# speedup vs baseline: 1.0136x; 1.0085x over previous
import functools

import jax
import jax.numpy as jnp
from jax import lax
from jax.experimental import pallas as pl
from jax.experimental.pallas import tpu as pltpu
from jax.experimental.pallas import tpu_sc as plsc

_B = 4 * 4096
_D = 1024
_NC = 2
_NS = 16
_NW = _NC * _NS
_BPW = _B // _NW
_C = 16
_NCHUNK = _BPW // _C
_NBUF = 3
_GDEPTH = 2


def _emb_body(idx_hbm, table_hbm, out_hbm, idx_v, *rest):
    bufs = rest[:_NBUF]
    gsems = rest[_NBUF:2 * _NBUF]
    osems = rest[2 * _NBUF:3 * _NBUF]
    wid = lax.axis_index("s") * _NC + lax.axis_index("c")
    base = wid * _BPW
    pltpu.sync_copy(idx_hbm.at[wid // 8, pl.ds((wid % 8) * _BPW, _BPW)], idx_v)

    ghandles = [None] * _NCHUNK
    ohandles = [None] * _NCHUNK

    def writeout(g):
        ghandles[g].wait()
        ohandles[g] = pltpu.async_copy(
            bufs[g % _NBUF], out_hbm.at[pl.ds(base + g * _C, _C)],
            osems[g % _NBUF])

    for c in range(_NCHUNK):
        if c >= _NBUF:
            ohandles[c - _NBUF].wait()
        ghandles[c] = pltpu.async_copy(
            table_hbm.at[idx_v.at[pl.ds(c * _C, _C)]], bufs[c % _NBUF],
            gsems[c % _NBUF])
        if c >= _GDEPTH - 1:
            writeout(c - (_GDEPTH - 1))
    for g in range(_NCHUNK - (_GDEPTH - 1), _NCHUNK):
        writeout(g)
    for g in range(_NCHUNK - _NBUF, _NCHUNK):
        ohandles[g].wait()


@functools.partial(jax.jit, static_argnames=())
def kernel(input_ids, word_embeddings):
    idx = input_ids
    mesh = plsc.VectorSubcoreMesh(core_axis_name="c", subcore_axis_name="s")
    run = pl.kernel(
        _emb_body,
        out_type=jax.ShapeDtypeStruct((_B, _D), jnp.float32),
        mesh=mesh,
        scratch_types=(
            [pltpu.VMEM((_BPW,), jnp.int32)]
            + [pltpu.VMEM((_C, _D), jnp.float32)] * _NBUF
            + [pltpu.SemaphoreType.DMA] * (2 * _NBUF)
        ),
    )
    out = run(idx, word_embeddings)
    return out.reshape(input_ids.shape + (_D,))


# speedup vs baseline: 1.0260x; 1.0123x over previous
import functools

import jax
import jax.numpy as jnp
from jax import lax
from jax.experimental import pallas as pl
from jax.experimental.pallas import tpu as pltpu
from jax.experimental.pallas import tpu_sc as plsc

_B = 4 * 4096
_D = 1024
_NC = 2
_NS = 16
_NW = _NC * _NS
_BPW = _B // _NW
_C = 16
_NCHUNK = _BPW // _C
_NBUF = 6
_GDEPTH = 4


def _emb_body(idx_hbm, table_hbm, out_hbm, idx_v, *rest):
    bufs = rest[:_NBUF]
    gsems = rest[_NBUF:2 * _NBUF]
    osems = rest[2 * _NBUF:3 * _NBUF]
    wid = lax.axis_index("s") * _NC + lax.axis_index("c")
    base = wid * _BPW
    pltpu.sync_copy(idx_hbm.at[wid // 8, pl.ds((wid % 8) * _BPW, _BPW)], idx_v)

    ghandles = [None] * _NCHUNK
    ohandles = [None] * _NCHUNK

    def writeout(g):
        ghandles[g].wait()
        ohandles[g] = pltpu.async_copy(
            bufs[g % _NBUF], out_hbm.at[pl.ds(base + g * _C, _C)],
            osems[g % _NBUF])

    for c in range(_NCHUNK):
        if c >= _NBUF:
            ohandles[c - _NBUF].wait()
        ghandles[c] = pltpu.async_copy(
            table_hbm.at[idx_v.at[pl.ds(c * _C, _C)]], bufs[c % _NBUF],
            gsems[c % _NBUF])
        if c >= _GDEPTH - 1:
            writeout(c - (_GDEPTH - 1))
    for g in range(_NCHUNK - (_GDEPTH - 1), _NCHUNK):
        writeout(g)
    for g in range(_NCHUNK - _NBUF, _NCHUNK):
        ohandles[g].wait()


@functools.partial(jax.jit, static_argnames=())
def kernel(input_ids, word_embeddings):
    idx = input_ids
    mesh = plsc.VectorSubcoreMesh(core_axis_name="c", subcore_axis_name="s")
    run = pl.kernel(
        _emb_body,
        out_type=jax.ShapeDtypeStruct((_B, _D), jnp.float32),
        mesh=mesh,
        scratch_types=(
            [pltpu.VMEM((_BPW,), jnp.int32)]
            + [pltpu.VMEM((_C, _D), jnp.float32)] * _NBUF
            + [pltpu.SemaphoreType.DMA] * (2 * _NBUF)
        ),
    )
    out = run(idx, word_embeddings)
    return out.reshape(input_ids.shape + (_D,))


# speedup vs baseline: 1.0323x; 1.0061x over previous
import functools

import jax
import jax.numpy as jnp
from jax import lax
from jax.experimental import pallas as pl
from jax.experimental.pallas import tpu as pltpu
from jax.experimental.pallas import tpu_sc as plsc

_B = 4 * 4096
_D = 1024
_NC = 2
_NS = 16
_NW = _NC * _NS
_BPW = _B // _NW
_C = 16
_NCHUNK = _BPW // _C
_NBUF = 7
_GDEPTH = 5


def _emb_body(idx_hbm, table_hbm, out_hbm, idx_v, *rest):
    bufs = rest[:_NBUF]
    gsems = rest[_NBUF:2 * _NBUF]
    osems = rest[2 * _NBUF:3 * _NBUF]
    wid = lax.axis_index("s") * _NC + lax.axis_index("c")
    base = wid * _BPW
    pltpu.sync_copy(idx_hbm.at[wid // 8, pl.ds((wid % 8) * _BPW, _BPW)], idx_v)

    ghandles = [None] * _NCHUNK
    ohandles = [None] * _NCHUNK

    def writeout(g):
        ghandles[g].wait()
        ohandles[g] = pltpu.async_copy(
            bufs[g % _NBUF], out_hbm.at[pl.ds(base + g * _C, _C)],
            osems[g % _NBUF])

    for c in range(_NCHUNK):
        if c >= _NBUF:
            ohandles[c - _NBUF].wait()
        ghandles[c] = pltpu.async_copy(
            table_hbm.at[idx_v.at[pl.ds(c * _C, _C)]], bufs[c % _NBUF],
            gsems[c % _NBUF])
        if c >= _GDEPTH - 1:
            writeout(c - (_GDEPTH - 1))
    for g in range(_NCHUNK - (_GDEPTH - 1), _NCHUNK):
        writeout(g)
    for g in range(_NCHUNK - _NBUF, _NCHUNK):
        ohandles[g].wait()


@functools.partial(jax.jit, static_argnames=())
def kernel(input_ids, word_embeddings):
    idx = input_ids
    mesh = plsc.VectorSubcoreMesh(core_axis_name="c", subcore_axis_name="s")
    run = pl.kernel(
        _emb_body,
        out_type=jax.ShapeDtypeStruct((_B, _D), jnp.float32),
        mesh=mesh,
        scratch_types=(
            [pltpu.VMEM((_BPW,), jnp.int32)]
            + [pltpu.VMEM((_C, _D), jnp.float32)] * _NBUF
            + [pltpu.SemaphoreType.DMA] * (2 * _NBUF)
        ),
    )
    out = run(idx, word_embeddings)
    return out.reshape(input_ids.shape + (_D,))
